# Initial kernel scaffold; baseline (speedup 1.0000x reference)
import functools

import jax
import jax.numpy as jnp
from jax import lax
from jax.experimental import pallas as pl
from jax.experimental.pallas import tpu as pltpu
from jax.experimental.pallas import tpu_sc as plsc

NX = 1000
XMIN, YMIN = 0.0, 0.0
DX = 0.001
DY = 0.001
NUM_GRIDS = NX * NX
DIM = 64

_info = plsc.get_sparse_core_info()
NC, NS, L = _info.num_cores, _info.num_subcores, _info.num_lanes
NW = NC * NS

CH = 512
GS = 128


@functools.partial(jax.jit, static_argnums=(0,))
def _grid_ebd_sc(B, t_flat, W):
    PW = B // NW
    NCH = PW // CH
    mesh = plsc.VectorSubcoreMesh(core_axis_name="c", subcore_axis_name="s")

    @functools.partial(
        pl.kernel,
        out_type=jax.ShapeDtypeStruct((B, DIM), jnp.float32),
        mesh=mesh,
        scratch_types=[
            pltpu.VMEM((2 * CH,), jnp.float32),
            pltpu.VMEM((CH,), jnp.int32),
            pltpu.VMEM((CH, DIM), jnp.float32),
            pltpu.SemaphoreType.DMA,
        ],
    )
    def k(t_hbm, w_hbm, out_hbm, t_v, idx_v, rows_v, sem):
        wid = lax.axis_index("s") * NC + lax.axis_index("c")
        base0 = wid * PW
        lane = lax.iota(jnp.int32, L)

        def body(c, carry):
            base = base0 + c * CH
            pltpu.sync_copy(t_hbm.at[pl.ds(2 * base, 2 * CH)], t_v)
            for s in range(CH // L):
                xi = 2 * (s * L + lane)
                x = plsc.load_gather(t_v, [xi])
                y = plsc.load_gather(t_v, [xi + 1])
                gx = ((x - XMIN) / DX).astype(jnp.int32)
                gy = (NX * ((y - YMIN) / DY)).astype(jnp.int32)
                g = gx + gy
                g = jnp.where((g > NUM_GRIDS) | (g < 0), NUM_GRIDS, g)
                idx_v[pl.ds(s * L, L)] = g
            copies = [
                pltpu.async_copy(
                    w_hbm.at[idx_v.at[pl.ds(j * GS, GS)]],
                    rows_v.at[pl.ds(j * GS, GS)],
                    sem,
                )
                for j in range(CH // GS)
            ]
            for cp in copies:
                cp.wait()
            pltpu.sync_copy(rows_v, out_hbm.at[pl.ds(base, CH)])
            return carry

        lax.fori_loop(0, NCH, body, 0)

    return k(t_flat, W)


def kernel(T, W):
    Bt, H, _ = T.shape
    B = Bt * H
    out = _grid_ebd_sc(B, T.reshape(-1), W)
    return out.reshape(Bt, H, DIM)


# speedup vs baseline: 1.0070x; 1.0070x over previous
import functools

import numpy as np
import jax
import jax.numpy as jnp
from jax import lax
from jax.experimental import pallas as pl
from jax.experimental.pallas import tpu as pltpu
from jax.experimental.pallas import tpu_sc as plsc

NX = 1000
NUM_GRIDS = NX * NX
DIM = 64
_DX32 = np.float32(1.0) / np.float32(1000)
_CX = np.float32(1.0 / float(_DX32))
_CY = np.float32(1000.0 / float(_DX32))

_info = plsc.get_sparse_core_info()
NC, NS, L = _info.num_cores, _info.num_subcores, _info.num_lanes
NW = NC * NS

CH = 512
GS = 128


@functools.partial(jax.jit, static_argnums=(0,))
def _grid_ebd_sc(B, t_flat, W):
    PW = B // NW
    NCH = PW // CH
    mesh = plsc.VectorSubcoreMesh(core_axis_name="c", subcore_axis_name="s")

    @functools.partial(
        pl.kernel,
        out_type=jax.ShapeDtypeStruct((B, DIM), jnp.float32),
        mesh=mesh,
        compiler_params=pltpu.CompilerParams(
            needs_layout_passes=False, use_tc_tiling_on_sc=False
        ),
        scratch_types=[
            pltpu.VMEM((2 * CH,), jnp.float32),
            pltpu.VMEM((CH,), jnp.int32),
            pltpu.VMEM((CH, DIM), jnp.float32),
            pltpu.SemaphoreType.DMA,
        ],
    )
    def k(t_hbm, w_hbm, out_hbm, t_v, idx_v, rows_v, sem):
        wid = lax.axis_index("s") * NC + lax.axis_index("c")
        base0 = wid * PW
        lane = lax.iota(jnp.int32, L)

        def body(c, carry):
            base = base0 + c * CH
            pltpu.sync_copy(t_hbm.at[pl.ds(2 * base, 2 * CH)], t_v)
            for s in range(CH // L):
                xi = 2 * (s * L + lane)
                x = plsc.load_gather(t_v, [xi])
                y = plsc.load_gather(t_v, [xi + 1])
                gx = (x * _CX).astype(jnp.int32)
                gy = (y * _CY).astype(jnp.int32)
                g = gx + gy
                g = jnp.where((g > NUM_GRIDS) | (g < 0), NUM_GRIDS, g)
                idx_v[pl.ds(s * L, L)] = g
            copies = [
                pltpu.async_copy(
                    w_hbm.at[idx_v.at[pl.ds(j * GS, GS)]],
                    rows_v.at[pl.ds(j * GS, GS)],
                    sem,
                )
                for j in range(CH // GS)
            ]
            for cp in copies:
                cp.wait()
            pltpu.sync_copy(rows_v, out_hbm.at[pl.ds(base, CH)])
            return carry

        lax.fori_loop(0, NCH, body, 0)

    return k(t_flat, W)


def kernel(T, W):
    Bt, H, _ = T.shape
    B = Bt * H
    out = _grid_ebd_sc(B, T.reshape(-1), W)
    return out.reshape(Bt, H, DIM)


# speedup vs baseline: 1.0391x; 1.0318x over previous
import functools

import numpy as np
import jax
import jax.numpy as jnp
from jax import lax
from jax.experimental import pallas as pl
from jax.experimental.pallas import tpu as pltpu
from jax.experimental.pallas import tpu_sc as plsc

NX = 1000
NUM_GRIDS = NX * NX
DIM = 64
_DX32 = np.float32(1.0) / np.float32(1000)
_CX = np.float32(1.0 / float(_DX32))
_CY = np.float32(1000.0 / float(_DX32))

_info = plsc.get_sparse_core_info()
NC, NS, L = _info.num_cores, _info.num_subcores, _info.num_lanes
NW = NC * NS

CH = 512
GS = 128


@functools.partial(jax.jit, static_argnums=(0,))
def _grid_ebd_sc(B, t_flat, W):
    PW = B // NW
    NCH = PW // CH
    mesh = plsc.VectorSubcoreMesh(core_axis_name="c", subcore_axis_name="s")

    @functools.partial(
        pl.kernel,
        out_type=jax.ShapeDtypeStruct((B, DIM), jnp.float32),
        mesh=mesh,
        compiler_params=pltpu.CompilerParams(
            needs_layout_passes=False, use_tc_tiling_on_sc=False
        ),
        scratch_types=[
            pltpu.VMEM((2 * CH,), jnp.float32),
            pltpu.VMEM((2 * CH,), jnp.float32),
            pltpu.VMEM((CH,), jnp.int32),
            pltpu.VMEM((CH,), jnp.int32),
            pltpu.VMEM((CH, DIM), jnp.float32),
            pltpu.VMEM((CH, DIM), jnp.float32),
            pltpu.SemaphoreType.DMA,
            pltpu.SemaphoreType.DMA,
            pltpu.SemaphoreType.DMA,
            pltpu.SemaphoreType.DMA,
            pltpu.SemaphoreType.DMA,
            pltpu.SemaphoreType.DMA,
        ],
    )
    def k(t_hbm, w_hbm, out_hbm, tv0, tv1, iv0, iv1, rv0, rv1,
          st0, st1, sg0, sg1, so0, so1):
        tv = (tv0, tv1)
        iv = (iv0, iv1)
        rv = (rv0, rv1)
        st = (st0, st1)
        sg = (sg0, sg1)
        so = (so0, so1)
        wid = lax.axis_index("s") * NC + lax.axis_index("c")
        base0 = wid * PW
        lane = lax.iota(jnp.int32, L)

        def tload(c, p):
            pltpu.async_copy(
                t_hbm.at[pl.ds(2 * (base0 + c * CH), 2 * CH)], tv[p], st[p]
            )

        def tdrain(p):
            pltpu.make_async_copy(
                t_hbm.at[pl.ds(0, 2 * CH)], tv[p], st[p]
            ).wait()

        def compute_idx(p):
            for s in range(CH // L):
                xi = 2 * (s * L + lane)
                x = plsc.load_gather(tv[p], [xi])
                y = plsc.load_gather(tv[p], [xi + 1])
                g = (x * _CX).astype(jnp.int32) + (y * _CY).astype(jnp.int32)
                g = jnp.where((g > NUM_GRIDS) | (g < 0), NUM_GRIDS, g)
                iv[p][pl.ds(s * L, L)] = g

        def fire_gathers(p):
            for j in range(CH // GS):
                pltpu.async_copy(
                    w_hbm.at[iv[p].at[pl.ds(j * GS, GS)]],
                    rv[p].at[pl.ds(j * GS, GS)],
                    sg[p],
                )

        def gdrain(p):
            pltpu.make_async_copy(w_hbm.at[pl.ds(0, CH)], rv[p], sg[p]).wait()

        def ostore(c, p):
            pltpu.async_copy(
                rv[p], out_hbm.at[pl.ds(base0 + c * CH, CH)], so[p]
            )

        def odrain(p):
            pltpu.make_async_copy(
                rv[p], out_hbm.at[pl.ds(0, CH)], so[p]
            ).wait()

        def step(c, p, first, last):
            tdrain(p)
            compute_idx(p)
            if not last:
                tload(c + 1, 1 - p)
            if not first:
                odrain(p)
            fire_gathers(p)
            gdrain(1 - p)
            ostore(c - 1, 1 - p)

        tload(0, 0)
        tdrain(0)
        compute_idx(0)
        tload(1, 1)
        fire_gathers(0)

        step(1, 1, first=True, last=False)

        def body(m, carry):
            c = 2 * m + 2
            step(c, 0, first=False, last=False)
            step(c + 1, 1, first=False, last=False)
            return carry

        lax.fori_loop(0, (NCH - 4) // 2, body, 0)

        step(NCH - 2, 0, first=False, last=False)
        step(NCH - 1, 1, first=False, last=True)

        gdrain(1)
        ostore(NCH - 1, 1)
        odrain(0)
        odrain(1)

    return k(t_flat, W)


def kernel(T, W):
    Bt, H, _ = T.shape
    B = Bt * H
    out = _grid_ebd_sc(B, T.reshape(-1), W)
    return out.reshape(Bt, H, DIM)


# speedup vs baseline: 1.2996x; 1.2507x over previous
import functools

import numpy as np
import jax
import jax.numpy as jnp
from jax import lax
from jax.experimental import pallas as pl
from jax.experimental.pallas import tpu as pltpu
from jax.experimental.pallas import tpu_sc as plsc

NX = 1000
NUM_GRIDS = NX * NX
DIM = 64
_DX32 = np.float32(1.0) / np.float32(1000)
_CX = np.float32(1.0 / float(_DX32))
_CY = np.float32(1000.0 / float(_DX32))

_info = plsc.get_sparse_core_info()
NC, NS, L = _info.num_cores, _info.num_subcores, _info.num_lanes
NW = NC * NS

BP = 128


@functools.partial(jax.jit, static_argnums=(0, 1))
def _grid_ebd_sc(Bt, H, t4, W):
    NBJ = Bt // BP
    NU = H * NBJ
    UW = NU // NW
    mesh = plsc.VectorSubcoreMesh(core_axis_name="c", subcore_axis_name="s")

    @functools.partial(
        pl.kernel,
        out_type=jax.ShapeDtypeStruct((H, DIM, Bt), jnp.float32),
        mesh=mesh,
        compiler_params=pltpu.CompilerParams(
            needs_layout_passes=False, use_tc_tiling_on_sc=False
        ),
        scratch_types=[
            pltpu.VMEM((2, BP), jnp.float32),
            pltpu.VMEM((2, BP), jnp.float32),
            pltpu.VMEM((BP,), jnp.int32),
            pltpu.VMEM((BP,), jnp.int32),
            pltpu.VMEM((BP, DIM), jnp.float32),
            pltpu.VMEM((BP, DIM), jnp.float32),
            pltpu.VMEM((DIM, BP), jnp.float32),
            pltpu.VMEM((DIM, BP), jnp.float32),
            pltpu.SemaphoreType.DMA,
            pltpu.SemaphoreType.DMA,
            pltpu.SemaphoreType.DMA,
            pltpu.SemaphoreType.DMA,
            pltpu.SemaphoreType.DMA,
            pltpu.SemaphoreType.DMA,
        ],
    )
    def k(t_hbm, w_hbm, out_hbm, tv0, tv1, iv0, iv1, rv0, rv1, ov0, ov1,
          st0, st1, sg0, sg1, so0, so1):
        tv = (tv0, tv1)
        iv = (iv0, iv1)
        rv = (rv0, rv1)
        ov = (ov0, ov1)
        st = (st0, st1)
        sg = (sg0, sg1)
        so = (so0, so1)
        wid = lax.axis_index("s") * NC + lax.axis_index("c")
        u0 = wid * UW
        lane = lax.iota(jnp.int32, L)
        lrow = [qq * L + lane for qq in range(4)]

        def hj(u):
            ug = u0 + u
            return ug // NBJ, ug % NBJ

        def tload(u, p):
            h, jb = hj(u)
            pltpu.async_copy(t_hbm.at[h, jb], tv[p], st[p])

        def tdrain(p):
            pltpu.make_async_copy(t_hbm.at[0, 0], tv[p], st[p]).wait()

        def compute_idx(p):
            for s in range(BP // L):
                x = tv[p][0, pl.ds(s * L, L)]
                y = tv[p][1, pl.ds(s * L, L)]
                g = (x * _CX).astype(jnp.int32) + (y * _CY).astype(jnp.int32)
                g = jnp.where((g > NUM_GRIDS) | (g < 0), NUM_GRIDS, g)
                iv[p][pl.ds(s * L, L)] = g

        def fire_gather(p):
            pltpu.async_copy(w_hbm.at[iv[p]], rv[p], sg[p])

        def gdrain(p):
            pltpu.make_async_copy(w_hbm.at[pl.ds(0, BP)], rv[p], sg[p]).wait()

        def transpose(p):
            def body_pt(pt, carry):
                ptv = lane * 0 + pt
                for qq in range(4):
                    v = rv[p][pt, pl.ds(qq * L, L)]
                    plsc.store_scatter(ov[p], [lrow[qq], ptv], v)
                return carry

            lax.fori_loop(0, BP, body_pt, 0)

        def ostore(u, p):
            h, jb = hj(u)
            pltpu.async_copy(
                ov[p], out_hbm.at[h, :, pl.ds(jb * BP, BP)], so[p]
            )

        def odrain(p):
            pltpu.make_async_copy(
                ov[p], out_hbm.at[0, :, pl.ds(0, BP)], so[p]
            ).wait()

        def step(u, p, do_odrain=True, do_tload=True):
            tdrain(p)
            compute_idx(p)
            if do_tload:
                tload(u + 1, 1 - p)
            fire_gather(p)
            gdrain(1 - p)
            if do_odrain:
                odrain(1 - p)
            transpose(1 - p)
            ostore(u - 1, 1 - p)

        tload(0, 0)
        tdrain(0)
        compute_idx(0)
        tload(1, 1)
        fire_gather(0)
        step(1, 1, do_odrain=False)
        step(2, 0, do_odrain=False)
        step(3, 1)

        def body(m, carry):
            u = 2 * m + 4
            step(u, 0)
            step(u + 1, 1)
            return carry

        lax.fori_loop(0, (UW - 6) // 2, body, 0)

        step(UW - 2, 0)
        step(UW - 1, 1, do_tload=False)

        gdrain(1)
        odrain(1)
        transpose(1)
        ostore(UW - 1, 1)
        odrain(0)
        odrain(1)

    return k(t4, W)


def kernel(T, W):
    Bt, H, _ = T.shape
    t4 = T.transpose(1, 0, 2).reshape(H, Bt // BP, BP, 2).transpose(0, 1, 3, 2)
    out3 = _grid_ebd_sc(Bt, H, t4, W)
    return out3.transpose(2, 0, 1)


# speedup vs baseline: 1.4750x; 1.1350x over previous
import functools

import numpy as np
import jax
import jax.numpy as jnp
from jax import lax
from jax.experimental import pallas as pl
from jax.experimental.pallas import tpu as pltpu
from jax.experimental.pallas import tpu_sc as plsc

NX = 1000
NUM_GRIDS = NX * NX
DIM = 64
_DX32 = np.float32(1.0) / np.float32(1000)
_CX = np.float32(1.0 / float(_DX32))
_CY = np.float32(1000.0 / float(_DX32))

_info = plsc.get_sparse_core_info()
NC, NS, L = _info.num_cores, _info.num_subcores, _info.num_lanes
NW = NC * NS

BP = 128


@functools.partial(jax.jit, static_argnums=(0, 1))
def _grid_ebd_sc(Bt, H, t4, W):
    NBJ = Bt // BP
    NU = H * NBJ
    UW = NU // NW
    mesh = plsc.VectorSubcoreMesh(core_axis_name="c", subcore_axis_name="s")

    @functools.partial(
        pl.kernel,
        out_type=jax.ShapeDtypeStruct((H, DIM // 8, Bt // BP, 8, BP), jnp.float32),
        mesh=mesh,
        compiler_params=pltpu.CompilerParams(
            needs_layout_passes=False, use_tc_tiling_on_sc=False
        ),
        scratch_types=[
            pltpu.VMEM((2, BP), jnp.float32),
            pltpu.VMEM((2, BP), jnp.float32),
            pltpu.VMEM((BP,), jnp.int32),
            pltpu.VMEM((BP,), jnp.int32),
            pltpu.VMEM((BP, DIM), jnp.float32),
            pltpu.VMEM((BP, DIM), jnp.float32),
            pltpu.VMEM((DIM // 8, 8, BP), jnp.float32),
            pltpu.VMEM((DIM // 8, 8, BP), jnp.float32),
            pltpu.SemaphoreType.DMA,
            pltpu.SemaphoreType.DMA,
            pltpu.SemaphoreType.DMA,
            pltpu.SemaphoreType.DMA,
            pltpu.SemaphoreType.DMA,
            pltpu.SemaphoreType.DMA,
        ],
    )
    def k(t_hbm, w_hbm, out_hbm, tv0, tv1, iv0, iv1, rv0, rv1, ov0, ov1,
          st0, st1, sg0, sg1, so0, so1):
        tv = (tv0, tv1)
        iv = (iv0, iv1)
        rv = (rv0, rv1)
        ov = (ov0, ov1)
        st = (st0, st1)
        sg = (sg0, sg1)
        so = (so0, so1)
        wid = lax.axis_index("s") * NC + lax.axis_index("c")
        u0 = wid * UW
        lane = lax.iota(jnp.int32, L)
        arow = [(qq * L + lane) // 8 for qq in range(4)]
        rrow = [(qq * L + lane) % 8 for qq in range(4)]

        def hj(u):
            ug = u0 + u
            return ug // NBJ, ug % NBJ

        def tload(u, p):
            h, jb = hj(u)
            pltpu.async_copy(t_hbm.at[h, jb], tv[p], st[p])

        def tdrain(p):
            pltpu.make_async_copy(t_hbm.at[0, 0], tv[p], st[p]).wait()

        def compute_idx(p):
            for s in range(BP // L):
                x = tv[p][0, pl.ds(s * L, L)]
                y = tv[p][1, pl.ds(s * L, L)]
                g = (x * _CX).astype(jnp.int32) + (y * _CY).astype(jnp.int32)
                g = jnp.where((g > NUM_GRIDS) | (g < 0), NUM_GRIDS, g)
                iv[p][pl.ds(s * L, L)] = g

        def fire_gather(p):
            pltpu.async_copy(w_hbm.at[iv[p]], rv[p], sg[p])

        def gdrain(p):
            pltpu.make_async_copy(w_hbm.at[pl.ds(0, BP)], rv[p], sg[p]).wait()

        def transpose(p):
            def body_pt(pt, carry):
                ptv = lane * 0 + pt
                for qq in range(4):
                    v = rv[p][pt, pl.ds(qq * L, L)]
                    plsc.store_scatter(ov[p], [arow[qq], rrow[qq], ptv], v)
                return carry

            lax.fori_loop(0, BP, body_pt, 0)

        def ostore(u, p):
            h, jb = hj(u)
            pltpu.async_copy(
                ov[p], out_hbm.at[h, :, jb, :, :], so[p]
            )

        def odrain(p):
            pltpu.make_async_copy(
                ov[p], out_hbm.at[0, :, 0, :, :], so[p]
            ).wait()

        def step(u, p, do_odrain=True, do_tload=True):
            tdrain(p)
            compute_idx(p)
            if do_tload:
                tload(u + 1, 1 - p)
            fire_gather(p)
            gdrain(1 - p)
            if do_odrain:
                odrain(1 - p)
            transpose(1 - p)
            ostore(u - 1, 1 - p)

        tload(0, 0)
        tdrain(0)
        compute_idx(0)
        tload(1, 1)
        fire_gather(0)
        step(1, 1, do_odrain=False)
        step(2, 0, do_odrain=False)
        step(3, 1)

        def body(m, carry):
            u = 2 * m + 4
            step(u, 0)
            step(u + 1, 1)
            return carry

        lax.fori_loop(0, (UW - 6) // 2, body, 0)

        step(UW - 2, 0)
        step(UW - 1, 1, do_tload=False)

        gdrain(1)
        odrain(1)
        transpose(1)
        ostore(UW - 1, 1)
        odrain(0)
        odrain(1)

    return k(t4, W)


def kernel(T, W):
    Bt, H, _ = T.shape
    t4 = T.transpose(1, 0, 2).reshape(H, Bt // BP, BP, 2).transpose(0, 1, 3, 2)
    out5 = _grid_ebd_sc(Bt, H, t4, W)
    return out5.transpose(2, 4, 0, 1, 3).reshape(Bt, H, DIM)
